# baseline (speedup 1.0000x reference)
import functools

import jax
import jax.numpy as jnp
from jax import lax
from jax.experimental import pallas as pl
from jax.experimental.pallas import tpu as pltpu
from jax.experimental.pallas import tpu_sc as plsc

N = 10000
NP = 10240
E = 320000
EP = 327680
EPT = EP // 32
D1 = 128
DH = 256
EPS = 1e-5
NPAIR = 16384

@functools.cache
def _mesh():
    return plsc.VectorSubcoreMesh(
        core_axis_name="c", subcore_axis_name="s", num_cores=2, num_subcores=16
    )


_SC_PARAMS = pltpu.CompilerParams(needs_layout_passes=False)


@functools.cache
def _deg_kernel():
    return functools.partial(
        pl.kernel,
        out_type=jax.ShapeDtypeStruct((2, 32, NP), jnp.float32),
        mesh=_mesh(),
        compiler_params=_SC_PARAMS,
        scratch_types=[
            pltpu.VMEM((EPT,), jnp.int32),
            pltpu.VMEM((EPT,), jnp.int32),
            pltpu.VMEM((NP,), jnp.float32),
            pltpu.VMEM((NP,), jnp.float32),
        ],
    )(_deg_body)


def _deg_body(ei, out, sidx, didx, hsrc, hdst):
    cid = lax.axis_index("c")
    sid = lax.axis_index("s")
    wid = sid * 2 + cid
    base = wid * EPT
    pltpu.sync_copy(ei.at[0, pl.ds(base, EPT)], sidx)
    pltpu.sync_copy(ei.at[1, pl.ds(base, EPT)], didx)
    zf = jnp.zeros((16,), jnp.float32)

    def zero(i, c):
        hsrc[pl.ds(i * 16, 16)] = zf
        hdst[pl.ds(i * 16, 16)] = zf
        return c

    lax.fori_loop(0, NP // 16, zero, 0)
    ones = jnp.ones((16,), jnp.float32)

    def acc(i, c):
        plsc.addupdate_scatter(hsrc, [sidx[pl.ds(i * 16, 16)]], ones)
        plsc.addupdate_scatter(hdst, [didx[pl.ds(i * 16, 16)]], ones)
        return c

    lax.fori_loop(0, EPT // 16, acc, 0)
    pltpu.sync_copy(hsrc, out.at[0, wid])
    pltpu.sync_copy(hdst, out.at[1, wid])


@functools.cache
def _make_agg_kernel(steps, split_cols, skew=(160, 0)):
    BLK = 8
    nblk = steps // BLK

    @functools.partial(
        pl.kernel,
        out_type=jax.ShapeDtypeStruct((2, NP, D1), jnp.float32),
        mesh=_mesh(),
        compiler_params=_SC_PARAMS,
        scratch_types=[
            pltpu.VMEM((2, 2, BLK, 128), jnp.int32),
            pltpu.VMEM((2, 128, D1), jnp.float32),
            pltpu.VMEM_SHARED((NP, D1), jnp.float32),
            pltpu.SemaphoreType.DMA((2,)),
            pltpu.SemaphoreType.DMA((2,)),
        ],
    )
    def _agg(tbl, ei, out, idxb, rows, acc, semg, semi):
        cid = lax.axis_index("c")
        sid = lax.axis_index("s")
        if split_cols:
            table = tbl.at[cid]
            rowbase = sid * steps
            npair = steps // (2 * BLK)
        else:
            q0, q1 = skew
            table = tbl
            rowbase = jnp.where(cid == 0, sid * q0, 16 * q0 + sid * q1)
            npair = jnp.where(cid == 0, q0 // (2 * BLK), q1 // (2 * BLK))
        nblk_c = npair * 2

        zf = jnp.zeros((16,), jnp.float32)

        def zero(i, c):
            rows[0, i // 8, pl.ds((i % 8) * 16, 16)] = zf
            return c

        lax.fori_loop(0, 128 * 8, zero, 0)
        for z in range(5):
            pltpu.sync_copy(
                rows.at[0], acc.at[pl.ds(sid * 640 + z * 128, 128)]
            )

        def fetch_stash(blkidx, sb):
            off = rowbase + jnp.minimum(blkidx, nblk_c - 1) * BLK
            return pltpu.async_copy(
                ei.at[:, pl.ds(off, BLK)], idxb.at[sb], semi.at[sb]
            )

        def wait_stash(sb):
            pltpu.make_async_copy(
                ei.at[:, pl.ds(0, BLK)], idxb.at[sb], semi.at[sb]
            ).wait()

        fetch_stash(jnp.int32(0), 0)
        plsc.subcore_barrier()

        def gath(sb, s, b):
            return pltpu.async_copy(
                table.at[idxb.at[sb, 0, s]], rows.at[b], semg.at[b]
            )

        def wait_gath(b):
            pltpu.make_async_copy(
                table.at[idxb.at[0, 0, 0]], rows.at[b], semg.at[b]
            ).wait()

        def pairblk(G, c):
            for sb in range(2):
                blkidx = G * 2 + sb
                wait_stash(sb)
                fetch_stash(blkidx + 1, 1 - sb)
                for s in range(0, BLK, 2):
                    gath(sb, s, 0)
                    gath(sb, s + 1, 1)
                    wait_gath(0)
                    pltpu.sync_copy(
                        rows.at[0], acc.at[idxb.at[sb, 1, s]], add=True
                    )
                    wait_gath(1)
                    pltpu.sync_copy(
                        rows.at[1], acc.at[idxb.at[sb, 1, s + 1]], add=True
                    )
            return c

        lax.fori_loop(0, npair, pairblk, 0)
        wait_stash(0)
        plsc.subcore_barrier()
        pltpu.sync_copy(
            acc.at[pl.ds(sid * 640, 640)], out.at[cid, pl.ds(sid * 640, 640)]
        )

    return _agg


@functools.cache
def _pair_gather():
    return functools.partial(
        pl.kernel,
        out_type=jax.ShapeDtypeStruct((4 * NPAIR, DH), jnp.float32),
        mesh=_mesh(),
        compiler_params=_SC_PARAMS,
        scratch_types=[
            pltpu.VMEM((16, 128), jnp.int32),
            pltpu.VMEM((2, 128, DH), jnp.float32),
            pltpu.SemaphoreType.DMA((2,)),
        ],
    )(_pair_gather_body)


def _pair_gather_body(h2, pidx, out, idxv, rows, sem):
    cid = lax.axis_index("c")
    sid = lax.axis_index("s")
    wid = sid * 2 + cid
    pltpu.sync_copy(pidx.at[pl.ds(wid * 16, 16)], idxv)

    def step(j, c):
        d = pltpu.async_copy(h2.at[idxv.at[j]], rows.at[0], sem.at[0])
        d.wait()
        pltpu.sync_copy(rows.at[0], out.at[pl.ds((wid * 16 + j) * 128, 128)])
        return c

    lax.fori_loop(0, 16, step, 0)


def _norms_body(degp_ref, norms_ref):
    deg = jnp.sum(degp_ref[...], axis=1)
    norms_ref[...] = jnp.where(deg > 0, lax.rsqrt(jnp.maximum(deg, 1.0)), 0.0)


def _scale_x_body(x_ref, norms_ref, xs_ref):
    xs_ref[...] = x_ref[...] * norms_ref[0]


def _layer1_body(aggp_ref, norms_ref, w1_ref, b1_ref, g_ref, be_ref, out_ref):
    agg = aggp_ref[0] + aggp_ref[1]
    aggn = agg * norms_ref[1]
    h = jnp.dot(aggn, w1_ref[...], preferred_element_type=jnp.float32)
    h = h + b1_ref[...]
    rowmask = lax.broadcasted_iota(jnp.int32, (NP, 1), 0) < N
    hm = jnp.where(rowmask, h, 0.0)
    mean = jnp.sum(hm, axis=0, keepdims=True) / N
    ex2 = jnp.sum(hm * hm, axis=0, keepdims=True) / N
    var = ex2 - mean * mean
    hbn = (h - mean) * lax.rsqrt(var + EPS) * g_ref[...] + be_ref[...]
    hr = jnp.maximum(hbn, 0.0)
    val = jnp.where(rowmask, hr * norms_ref[0], 0.0)
    out_ref[0] = val[:, :D1]
    out_ref[1] = val[:, D1:]


def _layer2_body(aggs_ref, norms_ref, w2_ref, b2_ref, out_ref):
    agg = jnp.concatenate([aggs_ref[0], aggs_ref[1]], axis=1)
    aggn = agg * norms_ref[1]
    out_ref[...] = (
        jnp.dot(aggn, w2_ref[...], preferred_element_type=jnp.float32) + b2_ref[...]
    )


def _pred_body(ga_ref, gb_ref, p1_ref, p1b_ref, p2_ref, p2b_ref, p3_ref, p3b_ref, out_ref):
    z = ga_ref[0] * gb_ref[0]
    z = jnp.maximum(jnp.dot(z, p1_ref[...], preferred_element_type=jnp.float32) + p1b_ref[...], 0.0)
    z = jnp.maximum(jnp.dot(z, p2_ref[...], preferred_element_type=jnp.float32) + p2b_ref[...], 0.0)
    out_ref[0] = jnp.dot(z, p3_ref[...], preferred_element_type=jnp.float32) + p3b_ref[...]


def kernel(x, edge_index, pos_edges, neg_edges, W1, b1, gamma, beta, W2, b2,
           P1w, P1b, P2w, P2b, P3w, P3b):
    ei = edge_index.astype(jnp.int32)
    ei_pad = jnp.concatenate([ei, jnp.full((2, EP - E), N, jnp.int32)], axis=1)
    ei3 = ei_pad.reshape(2, EP // 128, 128)
    xpad = jnp.concatenate([x, jnp.zeros((NP - N, D1), jnp.float32)], axis=0)
    pairs = jnp.concatenate(
        [pos_edges.astype(jnp.int32), neg_edges.astype(jnp.int32)], axis=0
    ).reshape(512, 128)

    degp = _deg_kernel()(ei_pad)

    norms = pl.pallas_call(
        _norms_body,
        out_shape=jax.ShapeDtypeStruct((2, NP), jnp.float32),
    )(degp)
    norms_c = norms[:, :, None]

    xs = pl.pallas_call(
        _scale_x_body,
        out_shape=jax.ShapeDtypeStruct((NP, D1), jnp.float32),
    )(xpad, norms_c)

    agg1p = _make_agg_kernel(80, False)(xs, ei3)

    h1s = pl.pallas_call(
        _layer1_body,
        out_shape=jax.ShapeDtypeStruct((2, NP, D1), jnp.float32),
    )(agg1p, norms_c, W1, b1.reshape(1, DH), gamma.reshape(1, DH),
      beta.reshape(1, DH))

    agg2s = _make_agg_kernel(160, True)(h1s, ei3)

    h2 = pl.pallas_call(
        _layer2_body,
        out_shape=jax.ShapeDtypeStruct((NP, DH), jnp.float32),
    )(agg2s, norms_c, W2, b2.reshape(1, DH))

    g = _pair_gather()(h2, pairs).reshape(4, NPAIR, DH)

    R = 2048
    pred = pl.pallas_call(
        _pred_body,
        grid=(2, NPAIR // R),
        in_specs=[
            pl.BlockSpec((1, R, DH), lambda i, r: (2 * i, r, 0)),
            pl.BlockSpec((1, R, DH), lambda i, r: (2 * i + 1, r, 0)),
            pl.BlockSpec((DH, DH), lambda i, r: (0, 0)),
            pl.BlockSpec((1, DH), lambda i, r: (0, 0)),
            pl.BlockSpec((DH, DH), lambda i, r: (0, 0)),
            pl.BlockSpec((1, DH), lambda i, r: (0, 0)),
            pl.BlockSpec((DH, 1), lambda i, r: (0, 0)),
            pl.BlockSpec((1, 1), lambda i, r: (0, 0)),
        ],
        out_specs=pl.BlockSpec((1, R, 1), lambda i, r: (i, r, 0)),
        out_shape=jax.ShapeDtypeStruct((2, NPAIR, 1), jnp.float32),
    )(g, g, P1w, P1b.reshape(1, DH), P2w, P2b.reshape(1, DH), P3w,
      P3b.reshape(1, 1))

    return (pred[0], pred[1])


# speedup vs baseline: 1.1540x; 1.1540x over previous
import functools

import jax
import jax.numpy as jnp
from jax import lax
from jax.experimental import pallas as pl
from jax.experimental.pallas import tpu as pltpu
from jax.experimental.pallas import tpu_sc as plsc

N = 10000
NP = 10240
E = 320000
EP = 327680
EPT = EP // 32
D1 = 128
DH = 256
EPS = 1e-5
NPAIR = 16384

@functools.cache
def _mesh():
    return plsc.VectorSubcoreMesh(
        core_axis_name="c", subcore_axis_name="s", num_cores=2, num_subcores=16
    )


_SC_PARAMS = pltpu.CompilerParams(needs_layout_passes=False)


@functools.cache
def _deg_kernel():
    return functools.partial(
        pl.kernel,
        out_type=jax.ShapeDtypeStruct((2, 32, NP), jnp.float32),
        mesh=_mesh(),
        compiler_params=_SC_PARAMS,
        scratch_types=[
            pltpu.VMEM((EPT,), jnp.int32),
            pltpu.VMEM((EPT,), jnp.int32),
            pltpu.VMEM((NP,), jnp.float32),
            pltpu.VMEM((NP,), jnp.float32),
        ],
    )(_deg_body)


def _deg_body(ei, out, sidx, didx, hsrc, hdst):
    cid = lax.axis_index("c")
    sid = lax.axis_index("s")
    wid = sid * 2 + cid
    base = wid * EPT
    pltpu.sync_copy(ei.at[0, pl.ds(base, EPT)], sidx)
    pltpu.sync_copy(ei.at[1, pl.ds(base, EPT)], didx)
    zf = jnp.zeros((16,), jnp.float32)

    def zero(i, c):
        hsrc[pl.ds(i * 16, 16)] = zf
        hdst[pl.ds(i * 16, 16)] = zf
        return c

    lax.fori_loop(0, NP // 16, zero, 0)
    ones = jnp.ones((16,), jnp.float32)

    def acc(i, c):
        plsc.addupdate_scatter(hsrc, [sidx[pl.ds(i * 16, 16)]], ones)
        plsc.addupdate_scatter(hdst, [didx[pl.ds(i * 16, 16)]], ones)
        return c

    lax.fori_loop(0, EPT // 16, acc, 0)
    pltpu.sync_copy(hsrc, out.at[0, wid])
    pltpu.sync_copy(hdst, out.at[1, wid])


@functools.cache
def _make_agg_kernel(steps, split_cols, skew=(144, 16)):
    BLK = 8
    nblk = steps // BLK

    @functools.partial(
        pl.kernel,
        out_type=jax.ShapeDtypeStruct((2, NP, D1), jnp.float32),
        mesh=_mesh(),
        compiler_params=_SC_PARAMS,
        scratch_types=[
            pltpu.VMEM((2, 2, BLK, 128), jnp.int32),
            pltpu.VMEM((2, 128, D1), jnp.float32),
            pltpu.VMEM_SHARED((NP, D1), jnp.float32),
            pltpu.SemaphoreType.DMA((2,)),
            pltpu.SemaphoreType.DMA((2,)),
        ],
    )
    def _agg(tbl, ei, out, idxb, rows, acc, semg, semi):
        cid = lax.axis_index("c")
        sid = lax.axis_index("s")
        if split_cols:
            table = tbl.at[cid]
            rowbase = sid * steps
            npair = steps // (2 * BLK)
        else:
            q0, q1 = skew
            table = tbl
            rowbase = jnp.where(cid == 0, sid * q0, 16 * q0 + sid * q1)
            npair = jnp.where(cid == 0, q0 // (2 * BLK), q1 // (2 * BLK))
        nblk_c = npair * 2

        zf = jnp.zeros((16,), jnp.float32)

        def zero(i, c):
            rows[0, i // 8, pl.ds((i % 8) * 16, 16)] = zf
            return c

        lax.fori_loop(0, 128 * 8, zero, 0)
        for z in range(5):
            pltpu.sync_copy(
                rows.at[0], acc.at[pl.ds(sid * 640 + z * 128, 128)]
            )

        def fetch_stash(blkidx, sb):
            off = rowbase + jnp.minimum(blkidx, nblk_c - 1) * BLK
            return pltpu.async_copy(
                ei.at[:, pl.ds(off, BLK)], idxb.at[sb], semi.at[sb]
            )

        def wait_stash(sb):
            pltpu.make_async_copy(
                ei.at[:, pl.ds(0, BLK)], idxb.at[sb], semi.at[sb]
            ).wait()

        fetch_stash(jnp.int32(0), 0)
        plsc.subcore_barrier()

        def gath(sb, s, b):
            return pltpu.async_copy(
                table.at[idxb.at[sb, 0, s]], rows.at[b], semg.at[b]
            )

        def wait_gath(b):
            pltpu.make_async_copy(
                table.at[idxb.at[0, 0, 0]], rows.at[b], semg.at[b]
            ).wait()

        def pairblk(G, c):
            for sb in range(2):
                blkidx = G * 2 + sb
                wait_stash(sb)
                fetch_stash(blkidx + 1, 1 - sb)
                for s in range(0, BLK, 2):
                    gath(sb, s, 0)
                    gath(sb, s + 1, 1)
                    wait_gath(0)
                    pltpu.sync_copy(
                        rows.at[0], acc.at[idxb.at[sb, 1, s]], add=True
                    )
                    wait_gath(1)
                    pltpu.sync_copy(
                        rows.at[1], acc.at[idxb.at[sb, 1, s + 1]], add=True
                    )
            return c

        lax.fori_loop(0, npair, pairblk, 0)
        wait_stash(0)
        plsc.subcore_barrier()
        pltpu.sync_copy(
            acc.at[pl.ds(sid * 640, 640)], out.at[cid, pl.ds(sid * 640, 640)]
        )

    return _agg


@functools.cache
def _pair_gather():
    return functools.partial(
        pl.kernel,
        out_type=jax.ShapeDtypeStruct((4 * NPAIR, DH), jnp.float32),
        mesh=_mesh(),
        compiler_params=_SC_PARAMS,
        scratch_types=[
            pltpu.VMEM((16, 128), jnp.int32),
            pltpu.VMEM((2, 128, DH), jnp.float32),
            pltpu.SemaphoreType.DMA((2,)),
        ],
    )(_pair_gather_body)


def _pair_gather_body(h2, pidx, out, idxv, rows, sem):
    cid = lax.axis_index("c")
    sid = lax.axis_index("s")
    wid = sid * 2 + cid
    pltpu.sync_copy(pidx.at[pl.ds(wid * 16, 16)], idxv)

    def step(j, c):
        d = pltpu.async_copy(h2.at[idxv.at[j]], rows.at[0], sem.at[0])
        d.wait()
        pltpu.sync_copy(rows.at[0], out.at[pl.ds((wid * 16 + j) * 128, 128)])
        return c

    lax.fori_loop(0, 16, step, 0)


def _norms_body(degp_ref, norms_ref):
    deg = jnp.sum(degp_ref[...], axis=1)
    norms_ref[...] = jnp.where(deg > 0, lax.rsqrt(jnp.maximum(deg, 1.0)), 0.0)


def _scale_x_body(x_ref, norms_ref, xs_ref):
    xs_ref[...] = x_ref[...] * norms_ref[0]


def _layer1_body(aggp_ref, norms_ref, w1_ref, b1_ref, g_ref, be_ref, out_ref):
    agg = aggp_ref[0] + aggp_ref[1]
    aggn = agg * norms_ref[1]
    h = jnp.dot(aggn, w1_ref[...], preferred_element_type=jnp.float32)
    h = h + b1_ref[...]
    rowmask = lax.broadcasted_iota(jnp.int32, (NP, 1), 0) < N
    hm = jnp.where(rowmask, h, 0.0)
    mean = jnp.sum(hm, axis=0, keepdims=True) / N
    ex2 = jnp.sum(hm * hm, axis=0, keepdims=True) / N
    var = ex2 - mean * mean
    hbn = (h - mean) * lax.rsqrt(var + EPS) * g_ref[...] + be_ref[...]
    hr = jnp.maximum(hbn, 0.0)
    val = jnp.where(rowmask, hr * norms_ref[0], 0.0)
    out_ref[0] = val[:, :D1]
    out_ref[1] = val[:, D1:]


def _layer2_body(aggs_ref, norms_ref, w2_ref, b2_ref, out_ref):
    agg = jnp.concatenate([aggs_ref[0], aggs_ref[1]], axis=1)
    aggn = agg * norms_ref[1]
    out_ref[...] = (
        jnp.dot(aggn, w2_ref[...], preferred_element_type=jnp.float32) + b2_ref[...]
    )


def _pred_body(ga_ref, gb_ref, p1_ref, p1b_ref, p2_ref, p2b_ref, p3_ref, p3b_ref, out_ref):
    z = ga_ref[0] * gb_ref[0]
    z = jnp.maximum(jnp.dot(z, p1_ref[...], preferred_element_type=jnp.float32) + p1b_ref[...], 0.0)
    z = jnp.maximum(jnp.dot(z, p2_ref[...], preferred_element_type=jnp.float32) + p2b_ref[...], 0.0)
    out_ref[0] = jnp.dot(z, p3_ref[...], preferred_element_type=jnp.float32) + p3b_ref[...]


def kernel(x, edge_index, pos_edges, neg_edges, W1, b1, gamma, beta, W2, b2,
           P1w, P1b, P2w, P2b, P3w, P3b):
    ei = edge_index.astype(jnp.int32)
    ei_pad = jnp.concatenate([ei, jnp.full((2, EP - E), N, jnp.int32)], axis=1)
    ei3 = ei_pad.reshape(2, EP // 128, 128)
    xpad = jnp.concatenate([x, jnp.zeros((NP - N, D1), jnp.float32)], axis=0)
    pairs = jnp.concatenate(
        [pos_edges.astype(jnp.int32), neg_edges.astype(jnp.int32)], axis=0
    ).reshape(512, 128)

    degp = _deg_kernel()(ei_pad)

    norms = pl.pallas_call(
        _norms_body,
        out_shape=jax.ShapeDtypeStruct((2, NP), jnp.float32),
    )(degp)
    norms_c = norms[:, :, None]

    xs = pl.pallas_call(
        _scale_x_body,
        out_shape=jax.ShapeDtypeStruct((NP, D1), jnp.float32),
    )(xpad, norms_c)

    agg1p = _make_agg_kernel(80, False)(xs, ei3)

    h1s = pl.pallas_call(
        _layer1_body,
        out_shape=jax.ShapeDtypeStruct((2, NP, D1), jnp.float32),
    )(agg1p, norms_c, W1, b1.reshape(1, DH), gamma.reshape(1, DH),
      beta.reshape(1, DH))

    agg2s = _make_agg_kernel(160, True)(h1s, ei3)

    h2 = pl.pallas_call(
        _layer2_body,
        out_shape=jax.ShapeDtypeStruct((NP, DH), jnp.float32),
    )(agg2s, norms_c, W2, b2.reshape(1, DH))

    g = _pair_gather()(h2, pairs).reshape(4, NPAIR, DH)

    R = 2048
    pred = pl.pallas_call(
        _pred_body,
        grid=(2, NPAIR // R),
        in_specs=[
            pl.BlockSpec((1, R, DH), lambda i, r: (2 * i, r, 0)),
            pl.BlockSpec((1, R, DH), lambda i, r: (2 * i + 1, r, 0)),
            pl.BlockSpec((DH, DH), lambda i, r: (0, 0)),
            pl.BlockSpec((1, DH), lambda i, r: (0, 0)),
            pl.BlockSpec((DH, DH), lambda i, r: (0, 0)),
            pl.BlockSpec((1, DH), lambda i, r: (0, 0)),
            pl.BlockSpec((DH, 1), lambda i, r: (0, 0)),
            pl.BlockSpec((1, 1), lambda i, r: (0, 0)),
        ],
        out_specs=pl.BlockSpec((1, R, 1), lambda i, r: (i, r, 0)),
        out_shape=jax.ShapeDtypeStruct((2, NPAIR, 1), jnp.float32),
    )(g, g, P1w, P1b.reshape(1, DH), P2w, P2b.reshape(1, DH), P3w,
      P3b.reshape(1, 1))

    return (pred[0], pred[1])


# speedup vs baseline: 1.1628x; 1.0076x over previous
import functools

import jax
import jax.numpy as jnp
from jax import lax
from jax.experimental import pallas as pl
from jax.experimental.pallas import tpu as pltpu
from jax.experimental.pallas import tpu_sc as plsc

N = 10000
NP = 10240
E = 320000
EP = 327680
EPT = EP // 32
D1 = 128
DH = 256
EPS = 1e-5
NPAIR = 16384

@functools.cache
def _mesh():
    return plsc.VectorSubcoreMesh(
        core_axis_name="c", subcore_axis_name="s", num_cores=2, num_subcores=16
    )


_SC_PARAMS = pltpu.CompilerParams(needs_layout_passes=False)


@functools.cache
def _deg_kernel():
    return functools.partial(
        pl.kernel,
        out_type=jax.ShapeDtypeStruct((2, 32, NP), jnp.float32),
        mesh=_mesh(),
        compiler_params=_SC_PARAMS,
        scratch_types=[
            pltpu.VMEM((EPT,), jnp.int32),
            pltpu.VMEM((EPT,), jnp.int32),
            pltpu.VMEM((NP,), jnp.float32),
            pltpu.VMEM((NP,), jnp.float32),
        ],
    )(_deg_body)


def _deg_body(ei, out, sidx, didx, hsrc, hdst):
    cid = lax.axis_index("c")
    sid = lax.axis_index("s")
    wid = sid * 2 + cid
    base = wid * EPT
    pltpu.sync_copy(ei.at[0, pl.ds(base, EPT)], sidx)
    pltpu.sync_copy(ei.at[1, pl.ds(base, EPT)], didx)
    zf = jnp.zeros((16,), jnp.float32)

    def zero(i, c):
        hsrc[pl.ds(i * 16, 16)] = zf
        hdst[pl.ds(i * 16, 16)] = zf
        return c

    lax.fori_loop(0, NP // 16, zero, 0)
    ones = jnp.ones((16,), jnp.float32)

    def acc(i, c):
        plsc.addupdate_scatter(hsrc, [sidx[pl.ds(i * 16, 16)]], ones)
        plsc.addupdate_scatter(hdst, [didx[pl.ds(i * 16, 16)]], ones)
        return c

    lax.fori_loop(0, EPT // 16, acc, 0)
    pltpu.sync_copy(hsrc, out.at[0, wid])
    pltpu.sync_copy(hdst, out.at[1, wid])


@functools.cache
def _make_agg_kernel(steps, split_cols, skew=(144, 16)):
    BLK = 8
    nblk = steps // BLK

    @functools.partial(
        pl.kernel,
        out_type=jax.ShapeDtypeStruct((2, NP, D1), jnp.float32),
        mesh=_mesh(),
        compiler_params=_SC_PARAMS,
        scratch_types=[
            pltpu.VMEM((2, 2, BLK, 128), jnp.int32),
            pltpu.VMEM((2, 128, D1), jnp.float32),
            pltpu.VMEM_SHARED((NP, D1), jnp.float32),
            pltpu.SemaphoreType.DMA((2,)),
            pltpu.SemaphoreType.DMA((2,)),
        ],
    )
    def _agg(tbl, ei, out, idxb, rows, acc, semg, semi):
        cid = lax.axis_index("c")
        sid = lax.axis_index("s")
        if split_cols:
            table = tbl.at[cid]
            rowbase = sid * steps
            npair = steps // (2 * BLK)
        else:
            q0, q1 = skew
            table = tbl
            rowbase = jnp.where(cid == 0, sid * q0, 16 * q0 + sid * q1)
            npair = jnp.where(cid == 0, q0 // (2 * BLK), q1 // (2 * BLK))
        nblk_c = npair * 2

        zf = jnp.zeros((16,), jnp.float32)

        def zero(i, c):
            rows[0, i // 8, pl.ds((i % 8) * 16, 16)] = zf
            return c

        lax.fori_loop(0, 128 * 8, zero, 0)
        for z in range(5):
            pltpu.sync_copy(
                rows.at[0], acc.at[pl.ds(sid * 640 + z * 128, 128)]
            )

        def fetch_stash(blkidx, sb):
            off = rowbase + jnp.minimum(blkidx, nblk_c - 1) * BLK
            return pltpu.async_copy(
                ei.at[:, pl.ds(off, BLK)], idxb.at[sb], semi.at[sb]
            )

        def wait_stash(sb):
            pltpu.make_async_copy(
                ei.at[:, pl.ds(0, BLK)], idxb.at[sb], semi.at[sb]
            ).wait()

        fetch_stash(jnp.int32(0), 0)
        plsc.subcore_barrier()

        def gath(sb, s, b):
            return pltpu.async_copy(
                table.at[idxb.at[sb, 0, s]], rows.at[b], semg.at[b]
            )

        def wait_gath(b):
            pltpu.make_async_copy(
                table.at[idxb.at[0, 0, 0]], rows.at[b], semg.at[b]
            ).wait()

        def pairblk(G, c):
            for sb in range(2):
                blkidx = G * 2 + sb
                wait_stash(sb)
                fetch_stash(blkidx + 1, 1 - sb)
                for s in range(0, BLK, 2):
                    gath(sb, s, 0)
                    gath(sb, s + 1, 1)
                    wait_gath(0)
                    pltpu.sync_copy(
                        rows.at[0], acc.at[idxb.at[sb, 1, s]], add=True
                    )
                    wait_gath(1)
                    pltpu.sync_copy(
                        rows.at[1], acc.at[idxb.at[sb, 1, s + 1]], add=True
                    )
            return c

        lax.fori_loop(0, npair, pairblk, 0)
        wait_stash(0)
        plsc.subcore_barrier()
        pltpu.sync_copy(
            acc.at[pl.ds(sid * 640, 640)], out.at[cid, pl.ds(sid * 640, 640)]
        )

    return _agg


@functools.cache
def _pair_gather():
    return functools.partial(
        pl.kernel,
        out_type=jax.ShapeDtypeStruct((4 * NPAIR, DH), jnp.float32),
        mesh=_mesh(),
        compiler_params=_SC_PARAMS,
        scratch_types=[
            pltpu.VMEM((16, 128), jnp.int32),
            pltpu.VMEM((2, 128, DH), jnp.float32),
            pltpu.SemaphoreType.DMA((2,)),
            pltpu.SemaphoreType.DMA((2,)),
        ],
    )(_pair_gather_body)


def _pair_gather_body(h2, pidx, out, idxv, rows, sem, semw):
    cid = lax.axis_index("c")
    sid = lax.axis_index("s")
    wid = sid * 2 + cid
    pltpu.sync_copy(pidx.at[pl.ds(wid * 16, 16)], idxv)

    def gath(j, b):
        return pltpu.async_copy(h2.at[idxv.at[j]], rows.at[b], sem.at[b])

    def wait_gath(b):
        pltpu.make_async_copy(h2.at[idxv.at[0]], rows.at[b], sem.at[b]).wait()

    def wr(j, b):
        return pltpu.async_copy(
            rows.at[b], out.at[pl.ds((wid * 16 + j) * 128, 128)], semw.at[b]
        )

    def wait_wr(b):
        pltpu.make_async_copy(
            rows.at[b], out.at[pl.ds(0, 128)], semw.at[b]
        ).wait()

    gath(0, 0)
    gath(1, 1)
    for j in range(16):
        b = j % 2
        wait_gath(b)
        wr(j, b)
        if j + 2 < 16:
            wait_wr(b)
            gath(j + 2, b)
    wait_wr(0)
    wait_wr(1)


def _norms_body(degp_ref, norms_ref):
    deg = jnp.sum(degp_ref[...], axis=1)
    norms_ref[...] = jnp.where(deg > 0, lax.rsqrt(jnp.maximum(deg, 1.0)), 0.0)


def _scale_x_body(x_ref, norms_ref, xs_ref):
    xs_ref[...] = x_ref[...] * norms_ref[0]


def _layer1_body(aggp_ref, norms_ref, w1_ref, b1_ref, g_ref, be_ref, out_ref):
    agg = aggp_ref[0] + aggp_ref[1]
    aggn = agg * norms_ref[1]
    h = jnp.dot(aggn, w1_ref[...], preferred_element_type=jnp.float32)
    h = h + b1_ref[...]
    rowmask = lax.broadcasted_iota(jnp.int32, (NP, 1), 0) < N
    hm = jnp.where(rowmask, h, 0.0)
    mean = jnp.sum(hm, axis=0, keepdims=True) / N
    ex2 = jnp.sum(hm * hm, axis=0, keepdims=True) / N
    var = ex2 - mean * mean
    hbn = (h - mean) * lax.rsqrt(var + EPS) * g_ref[...] + be_ref[...]
    hr = jnp.maximum(hbn, 0.0)
    val = jnp.where(rowmask, hr * norms_ref[0], 0.0)
    out_ref[0] = val[:, :D1]
    out_ref[1] = val[:, D1:]


def _layer2_body(aggs_ref, norms_ref, w2_ref, b2_ref, out_ref):
    agg = jnp.concatenate([aggs_ref[0], aggs_ref[1]], axis=1)
    aggn = agg * norms_ref[1]
    out_ref[...] = (
        jnp.dot(aggn, w2_ref[...], preferred_element_type=jnp.float32) + b2_ref[...]
    )


def _pred_body(ga_ref, gb_ref, p1_ref, p1b_ref, p2_ref, p2b_ref, p3_ref, p3b_ref, out_ref):
    z = ga_ref[0] * gb_ref[0]
    z = jnp.maximum(jnp.dot(z, p1_ref[...], preferred_element_type=jnp.float32) + p1b_ref[...], 0.0)
    z = jnp.maximum(jnp.dot(z, p2_ref[...], preferred_element_type=jnp.float32) + p2b_ref[...], 0.0)
    out_ref[0] = jnp.dot(z, p3_ref[...], preferred_element_type=jnp.float32) + p3b_ref[...]


def kernel(x, edge_index, pos_edges, neg_edges, W1, b1, gamma, beta, W2, b2,
           P1w, P1b, P2w, P2b, P3w, P3b):
    ei = edge_index.astype(jnp.int32)
    ei_pad = jnp.concatenate([ei, jnp.full((2, EP - E), N, jnp.int32)], axis=1)
    ei3 = ei_pad.reshape(2, EP // 128, 128)
    xpad = jnp.concatenate([x, jnp.zeros((NP - N, D1), jnp.float32)], axis=0)
    pairs = jnp.concatenate(
        [pos_edges.astype(jnp.int32), neg_edges.astype(jnp.int32)], axis=0
    ).reshape(512, 128)

    degp = _deg_kernel()(ei_pad)

    norms = pl.pallas_call(
        _norms_body,
        out_shape=jax.ShapeDtypeStruct((2, NP), jnp.float32),
    )(degp)
    norms_c = norms[:, :, None]

    xs = pl.pallas_call(
        _scale_x_body,
        out_shape=jax.ShapeDtypeStruct((NP, D1), jnp.float32),
    )(xpad, norms_c)

    agg1p = _make_agg_kernel(80, False)(xs, ei3)

    h1s = pl.pallas_call(
        _layer1_body,
        out_shape=jax.ShapeDtypeStruct((2, NP, D1), jnp.float32),
    )(agg1p, norms_c, W1, b1.reshape(1, DH), gamma.reshape(1, DH),
      beta.reshape(1, DH))

    agg2s = _make_agg_kernel(160, True)(h1s, ei3)

    h2 = pl.pallas_call(
        _layer2_body,
        out_shape=jax.ShapeDtypeStruct((NP, DH), jnp.float32),
    )(agg2s, norms_c, W2, b2.reshape(1, DH))

    g = _pair_gather()(h2, pairs).reshape(4, NPAIR, DH)

    R = 2048
    pred = pl.pallas_call(
        _pred_body,
        grid=(2, NPAIR // R),
        in_specs=[
            pl.BlockSpec((1, R, DH), lambda i, r: (2 * i, r, 0)),
            pl.BlockSpec((1, R, DH), lambda i, r: (2 * i + 1, r, 0)),
            pl.BlockSpec((DH, DH), lambda i, r: (0, 0)),
            pl.BlockSpec((1, DH), lambda i, r: (0, 0)),
            pl.BlockSpec((DH, DH), lambda i, r: (0, 0)),
            pl.BlockSpec((1, DH), lambda i, r: (0, 0)),
            pl.BlockSpec((DH, 1), lambda i, r: (0, 0)),
            pl.BlockSpec((1, 1), lambda i, r: (0, 0)),
        ],
        out_specs=pl.BlockSpec((1, R, 1), lambda i, r: (i, r, 0)),
        out_shape=jax.ShapeDtypeStruct((2, NPAIR, 1), jnp.float32),
    )(g, g, P1w, P1b.reshape(1, DH), P2w, P2b.reshape(1, DH), P3w,
      P3b.reshape(1, 1))

    return (pred[0], pred[1])
